# baseline (speedup 1.0000x reference)
import functools

import jax
import jax.numpy as jnp
from jax import lax
from jax.experimental import pallas as pl
from jax.experimental.pallas import tpu as pltpu
from jax.experimental.pallas import tpu_sc as plsc

TOP_K = 8


def _gates_chunk_body(x_ref, w_ref, gatesT_ref, imp_ref):
    i = pl.program_id(0)
    logitsT = jax.lax.dot_general(
        w_ref[...], x_ref[...],
        dimension_numbers=(((1,), (1,)), ((), ())),
        preferred_element_type=jnp.float32,
    )
    m = jnp.max(logitsT, axis=0, keepdims=True)
    e = jnp.exp(logitsT - m)
    s = jnp.sum(e, axis=0, keepdims=True)
    p = e / s
    num_e = p.shape[0]
    eio = lax.broadcasted_iota(jnp.int32, p.shape, 0)
    bits = lax.bitcast_convert_type(p, jnp.int32)
    gatesT_ref[...] = (bits & ~(num_e - 1)) | ((num_e - 1) - eio)

    @pl.when(i == 0)
    def _():
        imp_ref[...] = jnp.zeros_like(imp_ref)

    imp_ref[...] += jnp.sum(p, axis=1, keepdims=True)


def _gates_tail_body(x_ref, w_ref, imp0_ref, gatesT_ref, imp_ref, loss_ref):
    _gates_chunk_body(x_ref, w_ref, gatesT_ref, imp_ref)

    @pl.when(pl.program_id(0) == pl.num_programs(0) - 1)
    def _():
        imp = imp_ref[...] + imp0_ref[...]
        mean = jnp.mean(imp, axis=(0, 1), keepdims=True)
        std = jnp.sqrt(jnp.mean((imp - mean) ** 2, axis=(0, 1), keepdims=True))
        loss_ref[...] = (std / (mean + 1e-6)) ** 2


def _make_sc_topk(T, E, NW):
    NPT = T // NW
    NG = NPT // 16
    mesh = plsc.VectorSubcoreMesh(core_axis_name="c", subcore_axis_name="s")

    @functools.partial(
        pl.kernel,
        out_type=[
            jax.ShapeDtypeStruct((TOP_K, T), jnp.int32),
            jax.ShapeDtypeStruct((TOP_K, T), jnp.int32),
        ],
        mesh=mesh,
        scratch_types=[
            pltpu.VMEM((E, NPT), jnp.int32),
            pltpu.VMEM((TOP_K, NPT), jnp.int32),
            pltpu.VMEM((TOP_K, NPT), jnp.int32),
        ],
    )
    def sc_topk(gatesT_hbm, vals_hbm, idx_hbm, g_v, vstage, istage):
        wid = lax.axis_index("s") * 2 + lax.axis_index("c")
        base = wid * NPT
        pltpu.sync_copy(gatesT_hbm.at[:, pl.ds(base, NPT)], g_v)

        def group(gi, carry):
            off = gi * 16
            v = [jnp.full((16,), -1, jnp.int32) for _ in range(TOP_K)]
            for e in range(E):
                c = g_v[e, pl.ds(off, 16)]
                for j in range(TOP_K):
                    m = c > v[j]
                    vj = v[j]
                    v[j] = jnp.where(m, c, vj)
                    c = jnp.where(m, vj, c)
            for j in range(TOP_K):
                vstage[j, pl.ds(off, 16)] = v[j]
                istage[j, pl.ds(off, 16)] = (E - 1) - (v[j] & (E - 1))
            return carry

        lax.fori_loop(0, NG, group, 0)
        pltpu.sync_copy(vstage, vals_hbm.at[:, pl.ds(base, NPT)])
        pltpu.sync_copy(istage, idx_hbm.at[:, pl.ds(base, NPT)])

    return sc_topk


@functools.partial(jax.jit, static_argnames=("block_tokens", "tail_blocks"))
def _router(x, W, block_tokens=1024, tail_blocks=4):
    T, D = x.shape
    E = W.shape[0]
    BT = min(block_tokens, T)
    nblk_total = T // BT
    nblk0 = nblk_total - tail_blocks
    CT0 = nblk0 * BT
    CT1 = tail_blocks * BT
    gatesT0, imp0 = pl.pallas_call(
        _gates_chunk_body,
        grid=(nblk0,),
        in_specs=[
            pl.BlockSpec((BT, D), lambda i: (i, 0)),
            pl.BlockSpec((E, D), lambda i: (0, 0)),
        ],
        out_specs=[
            pl.BlockSpec((E, BT), lambda i: (0, i)),
            pl.BlockSpec((E, 1), lambda i: (0, 0)),
        ],
        out_shape=[
            jax.ShapeDtypeStruct((E, CT0), jnp.int32),
            jax.ShapeDtypeStruct((E, 1), jnp.float32),
        ],
    )(x, W)
    vb0, ix0 = _make_sc_topk(CT0, E, 32)(gatesT0)
    gatesT1, _, loss = pl.pallas_call(
        _gates_tail_body,
        grid=(tail_blocks,),
        in_specs=[
            pl.BlockSpec((BT, D), lambda i, b=nblk0: (b + i, 0)),
            pl.BlockSpec((E, D), lambda i: (0, 0)),
            pl.BlockSpec((E, 1), lambda i: (0, 0)),
        ],
        out_specs=[
            pl.BlockSpec((E, BT), lambda i: (0, i)),
            pl.BlockSpec((E, 1), lambda i: (0, 0)),
            pl.BlockSpec((1, 1), lambda i: (0, 0)),
        ],
        out_shape=[
            jax.ShapeDtypeStruct((E, CT1), jnp.int32),
            jax.ShapeDtypeStruct((E, 1), jnp.float32),
            jax.ShapeDtypeStruct((1, 1), jnp.float32),
        ],
    )(x, W, imp0)
    vb1, ix1 = _make_sc_topk(CT1, E, 32)(gatesT1)
    valsT_l, idxT_l = [vb0, vb1], [ix0, ix1]
    valsT_bits = jnp.concatenate(valsT_l, axis=1)
    idxT = jnp.concatenate(idxT_l, axis=1)
    vals = lax.bitcast_convert_type(valsT_bits.T, jnp.float32)
    return vals, idxT.T, loss[0, 0]


def kernel(x, W):
    return _router(x, W)


# speedup vs baseline: 1.1060x; 1.1060x over previous
import functools

import jax
import jax.numpy as jnp
from jax import lax
from jax.experimental import pallas as pl
from jax.experimental.pallas import tpu as pltpu
from jax.experimental.pallas import tpu_sc as plsc

TOP_K = 8


def _gates_chunk_body(x_ref, w_ref, gatesT_ref, imp_ref):
    i = pl.program_id(0)
    logitsT = jax.lax.dot_general(
        w_ref[...], x_ref[...],
        dimension_numbers=(((1,), (1,)), ((), ())),
        preferred_element_type=jnp.float32,
    )
    m = jnp.max(logitsT, axis=0, keepdims=True)
    e = jnp.exp(logitsT - m)
    s = jnp.sum(e, axis=0, keepdims=True)
    p = e / s
    num_e = p.shape[0]
    eio = lax.broadcasted_iota(jnp.int32, p.shape, 0)
    bits = lax.bitcast_convert_type(p, jnp.int32)
    gatesT_ref[...] = (bits & ~(num_e - 1)) | ((num_e - 1) - eio)

    @pl.when(i == 0)
    def _():
        imp_ref[...] = jnp.zeros_like(imp_ref)

    imp_ref[...] += jnp.sum(p, axis=1, keepdims=True)


def _loss_body(i0_ref, i1_ref, loss_ref):
    imp = i0_ref[...] + i1_ref[...]
    mean = jnp.mean(imp, axis=(0, 1), keepdims=True)
    std = jnp.sqrt(jnp.mean((imp - mean) ** 2, axis=(0, 1), keepdims=True))
    loss_ref[...] = (std / (mean + 1e-6)) ** 2


def _make_sc_topk(T, E, NW):
    NPT = T // NW
    NG = NPT // 16
    mesh = plsc.VectorSubcoreMesh(core_axis_name="c", subcore_axis_name="s")

    @functools.partial(
        pl.kernel,
        out_type=[
            jax.ShapeDtypeStruct((TOP_K, T), jnp.int32),
            jax.ShapeDtypeStruct((TOP_K, T), jnp.int32),
        ],
        mesh=mesh,
        scratch_types=[
            pltpu.VMEM((E, NPT), jnp.int32),
            pltpu.VMEM((TOP_K, NPT), jnp.int32),
            pltpu.VMEM((TOP_K, NPT), jnp.int32),
        ],
    )
    def sc_topk(gatesT_hbm, vals_hbm, idx_hbm, g_v, vstage, istage):
        wid = lax.axis_index("s") * 2 + lax.axis_index("c")
        base = wid * NPT
        pltpu.sync_copy(gatesT_hbm.at[:, pl.ds(base, NPT)], g_v)

        def group(gi, carry):
            off = gi * 16
            v = [jnp.full((16,), -1, jnp.int32) for _ in range(TOP_K)]
            for e in range(E):
                c = g_v[e, pl.ds(off, 16)]
                for j in range(TOP_K):
                    m = c > v[j]
                    vj = v[j]
                    v[j] = jnp.where(m, c, vj)
                    c = jnp.where(m, vj, c)
            for j in range(TOP_K):
                vstage[j, pl.ds(off, 16)] = v[j]
                istage[j, pl.ds(off, 16)] = (E - 1) - (v[j] & (E - 1))
            return carry

        lax.fori_loop(0, NG, group, 0)
        pltpu.sync_copy(vstage, vals_hbm.at[:, pl.ds(base, NPT)])
        pltpu.sync_copy(istage, idx_hbm.at[:, pl.ds(base, NPT)])

    return sc_topk


@functools.partial(jax.jit, static_argnames=("block_tokens", "tail_blocks"))
def _router(x, W, block_tokens=1024, tail_blocks=4):
    T, D = x.shape
    E = W.shape[0]
    BT = min(block_tokens, T)
    nblk_total = T // BT
    nblk0 = nblk_total - tail_blocks
    CT0 = nblk0 * BT
    CT1 = tail_blocks * BT
    gatesT0, imp0 = pl.pallas_call(
        _gates_chunk_body,
        grid=(nblk0,),
        in_specs=[
            pl.BlockSpec((BT, D), lambda i: (i, 0)),
            pl.BlockSpec((E, D), lambda i: (0, 0)),
        ],
        out_specs=[
            pl.BlockSpec((E, BT), lambda i: (0, i)),
            pl.BlockSpec((E, 1), lambda i: (0, 0)),
        ],
        out_shape=[
            jax.ShapeDtypeStruct((E, CT0), jnp.int32),
            jax.ShapeDtypeStruct((E, 1), jnp.float32),
        ],
    )(x, W)
    vb0, ix0 = _make_sc_topk(CT0, E, 32)(gatesT0)
    gatesT1, imp1 = pl.pallas_call(
        _gates_chunk_body,
        grid=(tail_blocks,),
        in_specs=[
            pl.BlockSpec((BT, D), lambda i, b=nblk0: (b + i, 0)),
            pl.BlockSpec((E, D), lambda i: (0, 0)),
        ],
        out_specs=[
            pl.BlockSpec((E, BT), lambda i: (0, i)),
            pl.BlockSpec((E, 1), lambda i: (0, 0)),
        ],
        out_shape=[
            jax.ShapeDtypeStruct((E, CT1), jnp.int32),
            jax.ShapeDtypeStruct((E, 1), jnp.float32),
        ],
    )(x, W)
    vb1, ix1 = _make_sc_topk(CT1, E, 32)(gatesT1)
    loss = pl.pallas_call(
        _loss_body,
        out_shape=jax.ShapeDtypeStruct((1, 1), jnp.float32),
    )(imp0, imp1)
    valsT_l, idxT_l = [vb0, vb1], [ix0, ix1]
    valsT_bits = jnp.concatenate(valsT_l, axis=1)
    idxT = jnp.concatenate(idxT_l, axis=1)
    vals = lax.bitcast_convert_type(valsT_bits.T, jnp.float32)
    return vals, idxT.T, loss[0, 0]


def kernel(x, W):
    return _router(x, W)


# speedup vs baseline: 1.1128x; 1.0061x over previous
import functools

import jax
import jax.numpy as jnp
from jax import lax
from jax.experimental import pallas as pl
from jax.experimental.pallas import tpu as pltpu
from jax.experimental.pallas import tpu_sc as plsc

TOP_K = 8


def _gates_chunk_body(x_ref, w_ref, gatesT_ref, imp_ref):
    i = pl.program_id(0)
    logitsT = jax.lax.dot_general(
        w_ref[...], x_ref[...],
        dimension_numbers=(((1,), (1,)), ((), ())),
        preferred_element_type=jnp.float32,
    )
    m = jnp.max(logitsT, axis=0, keepdims=True)
    e = jnp.exp(logitsT - m)
    s = jnp.sum(e, axis=0, keepdims=True)
    p = e / s
    num_e = p.shape[0]
    eio = lax.broadcasted_iota(jnp.int32, p.shape, 0)
    bits = lax.bitcast_convert_type(p, jnp.int32)
    gatesT_ref[...] = (bits & ~(num_e - 1)) | ((num_e - 1) - eio)

    @pl.when(i == 0)
    def _():
        imp_ref[...] = jnp.zeros_like(imp_ref)

    imp_ref[...] += jnp.sum(p, axis=1, keepdims=True)


def _loss_body(i0_ref, i1_ref, loss_ref):
    imp = i0_ref[...] + i1_ref[...]
    mean = jnp.mean(imp, axis=(0, 1), keepdims=True)
    std = jnp.sqrt(jnp.mean((imp - mean) ** 2, axis=(0, 1), keepdims=True))
    loss_ref[...] = (std / (mean + 1e-6)) ** 2


def _make_sc_topk(T, E, NW):
    NPT = T // NW
    NG = NPT // 16
    mesh = plsc.VectorSubcoreMesh(core_axis_name="c", subcore_axis_name="s")

    @functools.partial(
        pl.kernel,
        out_type=[
            jax.ShapeDtypeStruct((TOP_K, T), jnp.int32),
            jax.ShapeDtypeStruct((TOP_K, T), jnp.int32),
        ],
        mesh=mesh,
        scratch_types=[
            pltpu.VMEM((E, NPT), jnp.int32),
            pltpu.VMEM((TOP_K, NPT), jnp.int32),
            pltpu.VMEM((TOP_K, NPT), jnp.int32),
        ],
    )
    def sc_topk(gatesT_hbm, vals_hbm, idx_hbm, g_v, vstage, istage):
        wid = lax.axis_index("s") * 2 + lax.axis_index("c")
        base = wid * NPT
        pltpu.sync_copy(gatesT_hbm.at[:, pl.ds(base, NPT)], g_v)

        def group(gi, carry):
            off = gi * 16
            v = [jnp.full((16,), -1, jnp.int32) for _ in range(TOP_K)]
            for e in range(E):
                c = g_v[e, pl.ds(off, 16)]
                for j in range(TOP_K):
                    m = c > v[j]
                    vj = v[j]
                    v[j] = jnp.where(m, c, vj)
                    c = jnp.where(m, vj, c)
            for j in range(TOP_K):
                vstage[j, pl.ds(off, 16)] = v[j]
                istage[j, pl.ds(off, 16)] = (E - 1) - (v[j] & (E - 1))
            return carry

        lax.fori_loop(0, NG, group, 0)
        pltpu.sync_copy(vstage, vals_hbm.at[:, pl.ds(base, NPT)])
        pltpu.sync_copy(istage, idx_hbm.at[:, pl.ds(base, NPT)])

    return sc_topk


@functools.partial(jax.jit, static_argnames=("block_tokens", "tail_blocks"))
def _router(x, W, block_tokens=1024, tail_blocks=4):
    T, D = x.shape
    E = W.shape[0]
    BT = min(block_tokens, T)
    nblk_total = T // BT
    nblk0 = nblk_total - tail_blocks
    CT0 = nblk0 * BT
    CT1 = tail_blocks * BT
    gatesT0, imp0 = pl.pallas_call(
        _gates_chunk_body,
        grid=(nblk0,),
        in_specs=[
            pl.BlockSpec((BT, D), lambda i: (i, 0)),
            pl.BlockSpec((E, D), lambda i: (0, 0)),
        ],
        out_specs=[
            pl.BlockSpec((E, BT), lambda i: (0, i)),
            pl.BlockSpec((E, 1), lambda i: (0, 0)),
        ],
        out_shape=[
            jax.ShapeDtypeStruct((E, CT0), jnp.int32),
            jax.ShapeDtypeStruct((E, 1), jnp.float32),
        ],
    )(x, W)
    vb0, ix0 = _make_sc_topk(CT0, E, 32)(gatesT0)
    gatesT1, imp1 = pl.pallas_call(
        _gates_chunk_body,
        grid=(tail_blocks,),
        in_specs=[
            pl.BlockSpec((BT, D), lambda i, b=nblk0: (b + i, 0)),
            pl.BlockSpec((E, D), lambda i: (0, 0)),
        ],
        out_specs=[
            pl.BlockSpec((E, BT), lambda i: (0, i)),
            pl.BlockSpec((E, 1), lambda i: (0, 0)),
        ],
        out_shape=[
            jax.ShapeDtypeStruct((E, CT1), jnp.int32),
            jax.ShapeDtypeStruct((E, 1), jnp.float32),
        ],
    )(x, W)
    vb1, ix1 = _make_sc_topk(CT1, E, 32)(gatesT1)
    loss = pl.pallas_call(
        _loss_body,
        out_shape=jax.ShapeDtypeStruct((1, 1), jnp.float32),
    )(imp0, imp1)
    vals = jnp.concatenate(
        [lax.bitcast_convert_type(vb0.T, jnp.float32),
         lax.bitcast_convert_type(vb1.T, jnp.float32)], axis=0)
    idx = jnp.concatenate([ix0.T, ix1.T], axis=0)
    return vals, idx, loss[0, 0]


def kernel(x, W):
    return _router(x, W)
